# baseline (speedup 1.0000x reference)
import jax
import jax.numpy as jnp
from jax import lax
from jax.experimental import pallas as pl
from jax.experimental.pallas import tpu as pltpu
from jax.experimental.pallas import tpu_sc as plsc

N = 10000
D = 128
H = 64
A = 7
E = 320000

NC = 2
NS = 16
NW = NC * NS
CH = 125
CPS = 20

DNCH = 80
NPAD = 10240
DEG_PT = NPAD // NS

ANCH = 160
ROWS_PT = N // NS

_MESH = dict(core_axis_name="c", subcore_axis_name="s", num_cores=NC,
             num_subcores=NS)
_SC_PARAMS = pltpu.CompilerParams(use_tc_tiling_on_sc=False)


def _deg_body(col_hbm, out_hbm, col_slab, ones_v, zbuf, deg_acc):
    cc = lax.axis_index("c")
    sid = lax.axis_index("s")
    wid = cc * NS + sid

    @pl.loop(0, 8)
    def _(i):
        ones_v[pl.ds(i * 16, 16)] = jnp.ones((16,), jnp.float32)

    @pl.loop(0, DEG_PT // 16)
    def _(i):
        zbuf[pl.ds(i * 16, 16)] = jnp.zeros((16,), jnp.float32)

    pltpu.sync_copy(zbuf, deg_acc.at[pl.ds(sid * DEG_PT, DEG_PT)])
    pltpu.sync_copy(col_hbm.at[wid], col_slab)
    plsc.subcore_barrier()

    @pl.loop(0, DNCH)
    def _(j):
        pltpu.sync_copy(ones_v.at[pl.ds(0, CH)],
                        deg_acc.at[col_slab.at[j]], add=True)

    plsc.subcore_barrier()
    pltpu.sync_copy(deg_acc.at[pl.ds(sid * DEG_PT, DEG_PT)],
                    out_hbm.at[cc, pl.ds(sid * DEG_PT, DEG_PT)])


def _deg_call(col3):
    return pl.kernel(
        _deg_body,
        out_type=jax.ShapeDtypeStruct((NC, NPAD), jnp.float32),
        mesh=plsc.VectorSubcoreMesh(**_MESH),
        compiler_params=_SC_PARAMS,
        scratch_types=[
            pltpu.VMEM((DNCH, CH), jnp.int32),
            pltpu.VMEM((128,), jnp.float32),
            pltpu.VMEM((DEG_PT,), jnp.float32),
            pltpu.VMEM_SHARED((NPAD,), jnp.float32),
        ],
    )(col3)


def _mm_kernel(x_ref, w_ref, dinv_ref, hs_ref):
    h = jnp.dot(x_ref[...], w_ref[...], preferred_element_type=jnp.float32)
    hs = h * dinv_ref[...]
    hs_ref[0] = hs[:, :D]
    hs_ref[1] = hs[:, D:]


def _mm_call(x, Wcat, dinv):
    return pl.pallas_call(
        _mm_kernel,
        out_shape=jax.ShapeDtypeStruct((2, N, D), jnp.float32),
    )(x, Wcat, dinv)


def _agg_body(hs_hbm, row_hbm, col_hbm, out_hbm, row_slab, col_slab, gbuf,
              t_acc, sem0, sem1):
    cc = lax.axis_index("c")
    sid = lax.axis_index("s")
    sems = (sem0, sem1)
    g0 = gbuf.at[0]
    src = hs_hbm.at[cc]

    @pl.loop(0, CH)
    def _(r):
        @pl.loop(0, D // 16)
        def _(q):
            g0[r, pl.ds(q * 16, 16)] = jnp.zeros((16,), jnp.float32)

    @pl.loop(0, ROWS_PT // CH)
    def _(k):
        pltpu.sync_copy(g0, t_acc.at[pl.ds(sid * ROWS_PT + k * CH, CH)])

    plsc.subcore_barrier()

    @pl.loop(0, ANCH // CPS)
    def _(s):
        pltpu.sync_copy(row_hbm.at[sid, pl.ds(s * CPS, CPS)], row_slab)
        pltpu.sync_copy(col_hbm.at[sid, pl.ds(s * CPS, CPS)], col_slab)
        for b in range(2):
            pltpu.async_copy(src.at[row_slab.at[b]], gbuf.at[b], sems[b])

        @pl.loop(0, CPS // 2)
        def _(i):
            for b in range(2):
                j = 2 * i + b
                pltpu.make_async_copy(src.at[row_slab.at[j]], gbuf.at[b],
                                      sems[b]).wait()
                pltpu.sync_copy(gbuf.at[b], t_acc.at[col_slab.at[j]], add=True)

                @pl.when(j + 2 < CPS)
                def _():
                    pltpu.async_copy(src.at[row_slab.at[j + 2]],
                                     gbuf.at[b], sems[b])

    plsc.subcore_barrier()

    @pl.loop(0, ROWS_PT // CH)
    def _(k):
        off = sid * ROWS_PT + k * CH
        pltpu.sync_copy(t_acc.at[pl.ds(off, CH)], g0)
        pltpu.sync_copy(g0, out_hbm.at[cc, pl.ds(off, CH)])


def _agg_call(hs, row3, col3):
    return pl.kernel(
        _agg_body,
        out_type=jax.ShapeDtypeStruct((NC, N, D), jnp.float32),
        mesh=plsc.VectorSubcoreMesh(**_MESH),
        compiler_params=_SC_PARAMS,
        scratch_types=[
            pltpu.VMEM((CPS, CH), jnp.int32),
            pltpu.VMEM((CPS, CH), jnp.int32),
            pltpu.VMEM((2, CH, D), jnp.float32),
            pltpu.VMEM_SHARED((N, D), jnp.float32),
            pltpu.SemaphoreType.DMA,
            pltpu.SemaphoreType.DMA,
        ],
    )(hs, row3, col3)


def _tail_kernel(Ta_ref, Tc_ref, hsa_ref, hsc_ref, dinv_ref, bcat_ref,
                 med_ref, zc_scratch):
    za = (Ta_ref[...] + hsa_ref[...]) * dinv_ref[...]
    zcr = (Tc_ref[...] + hsc_ref[...]) * dinv_ref[...]
    zc = jnp.concatenate([za, zcr], axis=1) + bcat_ref[...]
    b = jax.lax.bitcast_convert_type(zc, jnp.int32)
    neg_mask = b >> 31
    ub = jax.lax.bitcast_convert_type(b, jnp.uint32)
    xor_val = jax.lax.bitcast_convert_type(neg_mask, jnp.uint32) | jnp.uint32(0x80000000)
    zc_scratch[...] = ub ^ xor_val
    zcu = zc_scratch[...]

    def body(i, carry):
        loA, hiA = carry
        midA = loA + (hiA - loA) // jnp.uint32(2)
        cA = jnp.sum((zcu <= midA).astype(jnp.int32), axis=0, keepdims=True)
        geA = cA >= 5000
        hiA = jnp.where(geA, midA, hiA)
        loA = jnp.where(geA, loA, midA + jnp.uint32(1))
        return loA, hiA

    lo0 = jnp.zeros((1, 2 * D), dtype=jnp.uint32)
    hi0 = jnp.full((1, 2 * D), 0xFFFFFFFF, dtype=jnp.uint32)
    loA, _ = lax.fori_loop(0, 32, body, (lo0, hi0))

    cnt = jnp.sum((zcu <= loA).astype(jnp.int32), axis=0, keepdims=True)
    zcs = jax.lax.bitcast_convert_type(zcu ^ jnp.uint32(0x80000000), jnp.int32)
    loS = jax.lax.bitcast_convert_type(loA ^ jnp.uint32(0x80000000), jnp.int32)
    big = jnp.int32(0x7FFFFFFF)
    above = jnp.where(zcs > loS, zcs, big)
    minS = jnp.min(above, axis=0, keepdims=True)
    minU = jax.lax.bitcast_convert_type(minS, jnp.uint32) ^ jnp.uint32(0x80000000)
    loB = jnp.where(cnt >= 5001, loA, minU)

    def u2f(u):
        is_pos = u >= jnp.uint32(0x80000000)
        ub2 = jnp.where(is_pos, u ^ jnp.uint32(0x80000000), ~u)
        return jax.lax.bitcast_convert_type(ub2, jnp.float32)

    med_ref[...] = 0.5 * (u2f(loA) + u2f(loB))


def _tail(Ta, Tc, hsa, hsc, dinv, bcat):
    return pl.pallas_call(
        _tail_kernel,
        out_shape=jax.ShapeDtypeStruct((1, 2 * D), jnp.float32),
        scratch_shapes=[pltpu.VMEM((N, 2 * D), jnp.uint32)],
    )(Ta, Tc, hsa, hsc, dinv, bcat)


def kernel(x, edge_index, action, aconv_W, aconv_b, a_W1, a_b1, a_W2, a_b2,
           a_W3, a_b3, cconv_W, cconv_b, c_W1, c_b1, c_W2, c_b2, c_W3, c_b3,
           actor_logstd):
    col3d = edge_index[1].reshape(NW, DNCH, CH)
    row3a = edge_index[0].reshape(NS, ANCH, CH)
    col3a = edge_index[1].reshape(NS, ANCH, CH)

    degp = _deg_call(col3d)
    deg = degp[0, :N] + degp[1, :N] + 1.0
    dinv = lax.rsqrt(deg).reshape(N, 1)

    Wcat = jnp.concatenate([aconv_W, cconv_W], axis=1)
    bcat = jnp.concatenate([aconv_b, cconv_b], axis=0).reshape(1, 2 * D)

    hs = _mm_call(x, Wcat, dinv)
    Tp = _agg_call(hs, row3a, col3a)

    med = _tail(Tp[0], Tp[1], hs[0], hs[1], dinv, bcat)
    ma = med[:, :D]
    mc = med[:, D:]
    t = jnp.tanh(ma @ a_W1 + a_b1)
    t = jnp.tanh(t @ a_W2 + a_b2)
    action_mean = t @ a_W3 + a_b3
    action_logstd = jnp.broadcast_to(actor_logstd, action_mean.shape)
    action_std = jnp.exp(action_logstd)
    log_prob = (-((action - action_mean) ** 2) / (2.0 * action_std ** 2)
                - action_logstd - 0.5 * jnp.log(2.0 * jnp.pi)).sum(axis=1)
    entropy = (0.5 + 0.5 * jnp.log(2.0 * jnp.pi) + action_logstd).sum(axis=1)
    tc = jnp.tanh(mc @ c_W1 + c_b1)
    tc = jnp.tanh(tc @ c_W2 + c_b2)
    value = tc @ c_W3 + c_b3
    return (action, log_prob, entropy, value)
